# Initial kernel scaffold; baseline (speedup 1.0000x reference)
import jax
import jax.numpy as jnp
from jax.experimental import pallas as pl


def kernel(x, edge_index, batch, convW1, convb1, bng1, bnb1, convW2, convb2, bns_g, bns_b, fcW, fcb):
    raise NotImplementedError("write your pallas kernel here")


# speedup vs baseline: 5.4209x; 5.4209x over previous
import functools

import jax
import jax.numpy as jnp
from jax import lax
from jax.experimental import pallas as pl
from jax.experimental.pallas import tpu as pltpu
from jax.experimental.pallas import tpu_sc as plsc

N = 10000
E = 320000
D = 128
C = 40
G = 128
L = 3

_NC, _NS = 2, 16
_NW = _NC * _NS
_CHUNK = 128
_NCHUNK = E // _CHUNK
_FULL = _NCHUNK // _NW
_TAIL = _NCHUNK - _FULL * _NW
_RPT = 624
_REM = N - _RPT * _NS
_ZROWS = 104


def _seg_body(src_hbm, dst_hbm, h_hbm, out0, out1,
              sidx, didx, rows, zbuf, acc, sem):
    cid = lax.axis_index("c")
    sid = lax.axis_index("s")
    wid = sid * _NC + cid

    zero = jnp.zeros((16,), jnp.float32)

    def _zb(i, carry):
        zbuf[i // 8, pl.ds((i % 8) * 16, 16)] = zero
        return carry

    lax.fori_loop(0, _ZROWS * (D // 16), _zb, 0)
    for r in range(_RPT // _ZROWS):
        pltpu.sync_copy(zbuf, acc.at[pl.ds(sid * _RPT + r * _ZROWS, _ZROWS)])

    @pl.when(sid == _NS - 1)
    def _():
        pltpu.sync_copy(zbuf.at[pl.ds(0, _REM)],
                        acc.at[pl.ds(_RPT * _NS, _REM)])

    plsc.subcore_barrier()

    def _chunk(base):
        pltpu.sync_copy(src_hbm.at[pl.ds(base, _CHUNK)], sidx)
        pltpu.sync_copy(dst_hbm.at[pl.ds(base, _CHUNK)], didx)
        pltpu.async_copy(h_hbm.at[sidx], rows, sem).wait()
        pltpu.sync_copy(rows, acc.at[didx], add=True)

    def _body(j, carry):
        _chunk((j * _NW + wid) * _CHUNK)
        return carry

    lax.fori_loop(0, _FULL, _body, 0)

    @pl.when(wid < _TAIL)
    def _():
        _chunk((_FULL * _NW + wid) * _CHUNK)

    plsc.subcore_barrier()

    @pl.when(cid == 0)
    def _():
        pltpu.sync_copy(acc.at[pl.ds(sid * _RPT, _RPT)],
                        out0.at[pl.ds(sid * _RPT, _RPT)])

        @pl.when(sid == _NS - 1)
        def _():
            pltpu.sync_copy(acc.at[pl.ds(_RPT * _NS, _REM)],
                            out0.at[pl.ds(_RPT * _NS, _REM)])

    @pl.when(cid == 1)
    def _():
        pltpu.sync_copy(acc.at[pl.ds(sid * _RPT, _RPT)],
                        out1.at[pl.ds(sid * _RPT, _RPT)])

        @pl.when(sid == _NS - 1)
        def _():
            pltpu.sync_copy(acc.at[pl.ds(_RPT * _NS, _REM)],
                            out1.at[pl.ds(_RPT * _NS, _REM)])


_seg_sum = functools.partial(
    pl.kernel,
    out_type=[jax.ShapeDtypeStruct((N, D), jnp.float32),
              jax.ShapeDtypeStruct((N, D), jnp.float32)],
    mesh=plsc.VectorSubcoreMesh(core_axis_name="c", subcore_axis_name="s"),
    scratch_types=[
        pltpu.VMEM((_CHUNK,), jnp.int32),
        pltpu.VMEM((_CHUNK,), jnp.int32),
        pltpu.VMEM((_CHUNK, D), jnp.float32),
        pltpu.VMEM((_ZROWS, D), jnp.float32),
        pltpu.VMEM_SHARED((N, D), jnp.float32),
        pltpu.SemaphoreType.DMA,
    ],
)(_seg_body)


_BLK = 2000
_HIGH = lax.Precision.HIGHEST


def _mlp_body(h_ref, a0_ref, a1_ref, w1_ref, b1_ref, g1_ref, t1_ref,
              w2_ref, b2_ref, g2_ref, t2_ref, fw_ref, yin_ref,
              hout_ref, yout_ref):
    z = h_ref[...] + a0_ref[...] + a1_ref[...]
    z = jnp.dot(z, w1_ref[...], preferred_element_type=jnp.float32,
                precision=_HIGH) + b1_ref[...]
    z = jnp.maximum(z * g1_ref[...] + t1_ref[...], 0.0)
    z = jnp.dot(z, w2_ref[...], preferred_element_type=jnp.float32,
                precision=_HIGH) + b2_ref[...]
    h2 = jnp.maximum(z * g2_ref[...] + t2_ref[...], 0.0)
    hout_ref[...] = h2
    yout_ref[...] = yin_ref[...] + jnp.dot(
        h2, fw_ref[...], preferred_element_type=jnp.float32, precision=_HIGH)


def _mlp(h, a0, a1, w1, b1, g1, t1, w2, b2, g2, t2, fw, yin):
    row = lambda i: (i, 0)
    full = lambda i: (0, 0)
    return pl.pallas_call(
        _mlp_body,
        grid=(N // _BLK,),
        in_specs=[
            pl.BlockSpec((_BLK, D), row),
            pl.BlockSpec((_BLK, D), row),
            pl.BlockSpec((_BLK, D), row),
            pl.BlockSpec((D, D), full),
            pl.BlockSpec((1, D), full),
            pl.BlockSpec((1, D), full),
            pl.BlockSpec((1, D), full),
            pl.BlockSpec((D, D), full),
            pl.BlockSpec((1, D), full),
            pl.BlockSpec((1, D), full),
            pl.BlockSpec((1, D), full),
            pl.BlockSpec((D, C), full),
            pl.BlockSpec((_BLK, C), row),
        ],
        out_specs=[pl.BlockSpec((_BLK, D), row),
                   pl.BlockSpec((_BLK, C), row)],
        out_shape=[jax.ShapeDtypeStruct((N, D), jnp.float32),
                   jax.ShapeDtypeStruct((N, C), jnp.float32)],
    )(h, a0, a1, w1, b1, g1, t1, w2, b2, g2, t2, fw, yin)


def _pool_body(x_ref, y_ref, b_ref, fw0_ref, fcb_ref, out_ref):
    i = pl.program_id(0)
    yb = y_ref[...] + jnp.dot(x_ref[...], fw0_ref[...],
                              preferred_element_type=jnp.float32,
                              precision=_HIGH)
    onehot = (b_ref[...] == lax.broadcasted_iota(jnp.int32, (1, G), 1)
              ).astype(jnp.float32)
    part = lax.dot_general(onehot, yb, (((0,), (0,)), ((), ())),
                           preferred_element_type=jnp.float32,
                           precision=_HIGH)

    @pl.when(i == 0)
    def _():
        out_ref[...] = part

    @pl.when(i > 0)
    def _():
        out_ref[...] += part

    @pl.when(i == pl.num_programs(0) - 1)
    def _():
        o = out_ref[...] + jnp.sum(fcb_ref[...], axis=0, keepdims=True)
        m = jnp.max(o, axis=-1, keepdims=True)
        lse = jnp.log(jnp.sum(jnp.exp(o - m), axis=-1, keepdims=True)) + m
        out_ref[...] = o - lse


def _pool(x, y, batch2, fw0, fcb):
    row = lambda i: (i, 0)
    full = lambda i: (0, 0)
    return pl.pallas_call(
        _pool_body,
        grid=(N // _BLK,),
        in_specs=[
            pl.BlockSpec((_BLK, D), row),
            pl.BlockSpec((_BLK, C), row),
            pl.BlockSpec((_BLK, 1), row),
            pl.BlockSpec((D, C), full),
            pl.BlockSpec((L + 1, C), full),
        ],
        out_specs=pl.BlockSpec((G, C), full),
        out_shape=jax.ShapeDtypeStruct((G, C), jnp.float32),
    )(x, y, batch2, fw0, fcb)


def kernel(x, edge_index, batch, convW1, convb1, bng1, bnb1,
           convW2, convb2, bns_g, bns_b, fcW, fcb):
    src = edge_index[0]
    dst = edge_index[1]
    batch2 = batch[:, None]
    h = x
    y = jnp.zeros((N, C), jnp.float32)
    for i in range(L):
        a0, a1 = _seg_sum(src, dst, h)
        h, y = _mlp(h, a0, a1,
                    convW1[i], convb1[i][None, :], bng1[i][None, :],
                    bnb1[i][None, :], convW2[i], convb2[i][None, :],
                    bns_g[i][None, :], bns_b[i][None, :], fcW[i + 1], y)
    return _pool(x, y, batch2, fcW[0], fcb)
